# baseline (device time: 99345 ns/iter reference)
import os

import jax
import jax.numpy as jnp
from jax import lax
from jax.experimental import pallas as pl
from jax.experimental.pallas import tpu as pltpu

_ABLATE = os.environ.get("KERNEL_ABLATE", "full")
_DO_COMPUTE = _ABLATE != "commonly"
_DO_RS = _ABLATE not in ("nocomm",)
_DO_AG = _ABLATE not in ("nocomm", "noag")

N_DEV = 4
N_TOK = 2048
D = 1024
H = 1024
E_LOCAL = 8
CAP = 128
SLOTS = E_LOCAL * CAP
CHUNK = N_TOK // N_DEV
HC = H // 2


def kernel(x, router_W, route_idx, expert_W, shared_W):
    my = lax.axis_index("i")
    first = my * E_LOCAL

    scores = x @ router_W
    s_max = jnp.max(scores, axis=-1, keepdims=True)
    p_top = 1.0 / jnp.sum(jnp.exp(scores - s_max), axis=-1)
    ridx = route_idx[:, 0]
    lej = ridx - first
    mine = (lej >= 0) & (lej < E_LOCAL)
    emask = (lej[:, None] == jnp.arange(E_LOCAL)[None, :])
    pos = jnp.cumsum(emask.astype(jnp.int32), axis=0) - 1
    pos_tok = jnp.sum(jnp.where(emask, pos, 0), axis=-1)
    col = jnp.where(mine & (pos_tok < CAP), lej * CAP + pos_tok, SLOTS)
    slot_iota = jnp.arange(SLOTS)
    onehot = col[:, None] == slot_iota[None, :]
    G = onehot.T.astype(jnp.bfloat16)
    Gtw = (onehot * p_top[:, None]).astype(jnp.bfloat16)

    def body(x_ref, G_ref, Gtw_ref, expert_W_ref, shared_W_ref,
             out_ref, wbuf, ys_ref, sbufR, sbufL, rsR, rsL,
             agR, agL, dma_sems, send_sems, recv_sems):
        my = lax.axis_index("i")
        left = lax.rem(my - 1 + N_DEV, N_DEV)
        right = lax.rem(my + 1, N_DEV)

        barrier_sem = pltpu.get_barrier_semaphore()
        for nbr in (left, right):
            pl.semaphore_signal(
                barrier_sem, inc=1,
                device_id=(nbr,), device_id_type=pl.DeviceIdType.MESH,
            )
        pl.semaphore_wait(barrier_sem, 2)

        pltpu.make_async_copy(expert_W_ref.at[0], wbuf.at[0],
                              dma_sems.at[0]).start()

        xb = x_ref[:, :].astype(jnp.bfloat16)

        xg = jnp.dot(G_ref[:, :], xb, preferred_element_type=jnp.float32)
        xgb = xg.astype(jnp.bfloat16)

        for j in range(E_LOCAL):
            if j + 1 < E_LOCAL:
                pltpu.make_async_copy(expert_W_ref.at[j + 1],
                                      wbuf.at[(j + 1) % 2],
                                      dma_sems.at[(j + 1) % 2]).start()
            pltpu.make_async_copy(expert_W_ref.at[j], wbuf.at[j % 2],
                                  dma_sems.at[j % 2]).wait()
            if _DO_COMPUTE:
                yj = jnp.dot(xgb[j * CAP:(j + 1) * CAP, :],
                             wbuf[j % 2].astype(jnp.bfloat16),
                             preferred_element_type=jnp.float32)
                ys_ref[j * CAP:(j + 1) * CAP, :] = yj.astype(jnp.bfloat16)
            else:
                ys_ref[j * CAP:(j + 1) * CAP, :] = (
                    wbuf[j % 2, :CAP, :].astype(jnp.bfloat16))
        ys = ys_ref[:, :]

        def partial_chunk(c, lo, width, with_shared=False):
            rows = pl.ds(c * CHUNK, CHUNK)
            acc = jnp.dot(Gtw_ref[rows, :], ys[:, lo:lo + width],
                          preferred_element_type=jnp.float32)
            if with_shared:
                acc += jnp.dot(x_ref[rows, :].astype(jnp.bfloat16),
                               shared_W_ref[:, lo:lo + width].astype(
                                   jnp.bfloat16),
                               preferred_element_type=jnp.float32)
            out_ref[rows, lo:lo + width] = acc

        def ring_rdma(src, dst, sem_idx, dst_dev):
            return pltpu.make_async_remote_copy(
                src_ref=src, dst_ref=dst,
                send_sem=send_sems.at[sem_idx],
                recv_sem=recv_sems.at[sem_idx],
                device_id=(dst_dev,),
                device_id_type=pl.DeviceIdType.MESH,
            )

        colR = pl.ds(0, HC)
        colL = pl.ds(HC, HC)
        c_m1 = lax.rem(my - 1 + N_DEV, N_DEV)
        c_p1 = lax.rem(my + 1, N_DEV)
        c_p2 = lax.rem(my + 2, N_DEV)

        def rs_start(s, c_sR, c_sL):
            sbufR[s] = out_ref[pl.ds(c_sR * CHUNK, CHUNK), colR].astype(
                jnp.bfloat16)
            sbufL[s] = out_ref[pl.ds(c_sL * CHUNK, CHUNK), colL].astype(
                jnp.bfloat16)
            rR = ring_rdma(sbufR.at[s], rsR.at[s], s, right)
            rL = ring_rdma(sbufL.at[s], rsL.at[s], 3 + s, left)
            rR.start()
            rL.start()
            return rR, rL

        def rs_finish(rR, rL, s, c_rR, c_rL):
            rR.wait()
            rL.wait()
            out_ref[pl.ds(c_rR * CHUNK, CHUNK), colR] += rsR[s].astype(
                jnp.float32)
            out_ref[pl.ds(c_rL * CHUNK, CHUNK), colL] += rsL[s].astype(
                jnp.float32)

        partial_chunk(my, 0, H, with_shared=True)

        if _DO_RS:
            r0 = rs_start(0, my, my)
            partial_chunk(c_m1, 0, HC)
            partial_chunk(c_p1, HC, HC)
            rs_finish(*r0, 0, c_m1, c_p1)

            r1 = rs_start(1, c_m1, c_p1)
            partial_chunk(c_p2, 0, H)
            rs_finish(*r1, 1, c_p2, c_p2)

            r2 = rs_start(2, c_p2, c_p2)
            partial_chunk(c_p1, 0, HC)
            partial_chunk(c_m1, HC, HC)
            rs_finish(*r2, 2, c_p1, c_m1)
        else:
            partial_chunk(c_m1, 0, H)
            partial_chunk(c_p1, 0, H)
            partial_chunk(c_p2, 0, H)

        if not _DO_AG:
            return
        agR[3] = out_ref[pl.ds(c_p1 * CHUNK, CHUNK), colR].astype(
            jnp.bfloat16)
        agL[3] = out_ref[pl.ds(c_m1 * CHUNK, CHUNK), colL].astype(
            jnp.bfloat16)
        for s in range(N_DEV - 1):
            c_rR = lax.rem(my - s + N_DEV, N_DEV)
            c_rL = lax.rem(my + s, N_DEV)
            src_slot = 3 if s == 0 else s - 1
            rR = ring_rdma(agR.at[src_slot], agR.at[s], 6 + s, right)
            rL = ring_rdma(agL.at[src_slot], agL.at[s], 9 + s, left)
            rR.start()
            rL.start()
            rR.wait()
            rL.wait()
            out_ref[pl.ds(c_rR * CHUNK, CHUNK), colR] = agR[s].astype(
                jnp.float32)
            out_ref[pl.ds(c_rL * CHUNK, CHUNK), colL] = agL[s].astype(
                jnp.float32)

    out_shape = jax.ShapeDtypeStruct((N_TOK, H), jnp.float32)
    return pl.pallas_call(
        body,
        out_shape=out_shape,
        in_specs=[
            pl.BlockSpec(memory_space=pltpu.VMEM),
            pl.BlockSpec(memory_space=pltpu.VMEM),
            pl.BlockSpec(memory_space=pltpu.VMEM),
            pl.BlockSpec(memory_space=pl.ANY),
            pl.BlockSpec(memory_space=pltpu.VMEM),
        ],
        out_specs=pl.BlockSpec(memory_space=pltpu.VMEM),
        scratch_shapes=[
            pltpu.VMEM((2, D, H), jnp.float32),
            pltpu.VMEM((SLOTS, H), jnp.bfloat16),
            pltpu.VMEM((3, CHUNK, HC), jnp.bfloat16),
            pltpu.VMEM((3, CHUNK, HC), jnp.bfloat16),
            pltpu.VMEM((3, CHUNK, HC), jnp.bfloat16),
            pltpu.VMEM((3, CHUNK, HC), jnp.bfloat16),
            pltpu.VMEM((4, CHUNK, HC), jnp.bfloat16),
            pltpu.VMEM((4, CHUNK, HC), jnp.bfloat16),
            pltpu.SemaphoreType.DMA((2,)),
            pltpu.SemaphoreType.DMA((12,)),
            pltpu.SemaphoreType.DMA((12,)),
        ],
        compiler_params=pltpu.CompilerParams(
            collective_id=0,
            vmem_limit_bytes=100 * 1024 * 1024,
        ),
    )(x, G, Gtw, expert_W, shared_W)


# device time: 95613 ns/iter; 1.0390x vs baseline; 1.0390x over previous
import os

import jax
import jax.numpy as jnp
from jax import lax
from jax.experimental import pallas as pl
from jax.experimental.pallas import tpu as pltpu

_ABLATE = os.environ.get("KERNEL_ABLATE", "full")
_DO_COMPUTE = _ABLATE != "commonly"
_DO_RS = _ABLATE not in ("nocomm",)
_DO_AG = _ABLATE not in ("nocomm", "noag")

N_DEV = 4
N_TOK = 2048
D = 1024
H = 1024
E_LOCAL = 8
CAP = 128
SLOTS = E_LOCAL * CAP
CHUNK = N_TOK // N_DEV
HC = H // 2


def kernel(x, router_W, route_idx, expert_W, shared_W):
    my = lax.axis_index("i")
    first = my * E_LOCAL

    scores = x @ router_W
    s_max = jnp.max(scores, axis=-1, keepdims=True)
    p_top = 1.0 / jnp.sum(jnp.exp(scores - s_max), axis=-1)
    ridx = route_idx[:, 0]
    lej = ridx - first
    mine = (lej >= 0) & (lej < E_LOCAL)
    emask = (lej[:, None] == jnp.arange(E_LOCAL)[None, :])
    pos = jnp.cumsum(emask.astype(jnp.int32), axis=0) - 1
    pos_tok = jnp.sum(jnp.where(emask, pos, 0), axis=-1)
    col = jnp.where(mine & (pos_tok < CAP), lej * CAP + pos_tok, SLOTS)
    col2d = col[:, None].astype(jnp.int32)
    colrow = col[None, :].astype(jnp.int32)
    pw2d = p_top[:, None].astype(jnp.float32)

    def body(x_ref, col_ref, colrow_ref, pw_ref, expert_W_ref, shared_W_ref,
             out_ref, wbuf, ys_ref, sbufR, sbufL, rsR, rsL,
             agR, agL, dma_sems, send_sems, recv_sems):
        my = lax.axis_index("i")
        left = lax.rem(my - 1 + N_DEV, N_DEV)
        right = lax.rem(my + 1, N_DEV)

        barrier_sem = pltpu.get_barrier_semaphore()
        for nbr in (left, right):
            pl.semaphore_signal(
                barrier_sem, inc=1,
                device_id=(nbr,), device_id_type=pl.DeviceIdType.MESH,
            )
        pl.semaphore_wait(barrier_sem, 2)

        pltpu.make_async_copy(expert_W_ref.at[0], wbuf.at[0],
                              dma_sems.at[0]).start()

        xb = x_ref[:, :].astype(jnp.bfloat16)

        slot_i = lax.broadcasted_iota(jnp.int32, (SLOTS, N_TOK), 0)
        G = (slot_i == colrow_ref[:, :]).astype(jnp.bfloat16)

        xg = jnp.dot(G, xb, preferred_element_type=jnp.float32)
        xgb = xg.astype(jnp.bfloat16)

        for j in range(E_LOCAL):
            if j + 1 < E_LOCAL:
                pltpu.make_async_copy(expert_W_ref.at[j + 1],
                                      wbuf.at[(j + 1) % 2],
                                      dma_sems.at[(j + 1) % 2]).start()
            pltpu.make_async_copy(expert_W_ref.at[j], wbuf.at[j % 2],
                                  dma_sems.at[j % 2]).wait()
            if _DO_COMPUTE:
                yj = jnp.dot(xgb[j * CAP:(j + 1) * CAP, :],
                             wbuf[j % 2].astype(jnp.bfloat16),
                             preferred_element_type=jnp.float32)
                ys_ref[j * CAP:(j + 1) * CAP, :] = yj.astype(jnp.bfloat16)
            else:
                ys_ref[j * CAP:(j + 1) * CAP, :] = (
                    wbuf[j % 2, :CAP, :].astype(jnp.bfloat16))
        ys = ys_ref[:, :]

        def partial_chunk(c, lo, width, with_shared=False):
            rows = pl.ds(c * CHUNK, CHUNK)
            slot_j = lax.broadcasted_iota(jnp.int32, (CHUNK, SLOTS), 1)
            Sw = jnp.where(slot_j == col_ref[rows, :], pw_ref[rows, :],
                           0.0).astype(jnp.bfloat16)
            acc = jnp.dot(Sw, ys[:, lo:lo + width],
                          preferred_element_type=jnp.float32)
            if with_shared:
                acc += jnp.dot(x_ref[rows, :].astype(jnp.bfloat16),
                               shared_W_ref[:, lo:lo + width].astype(
                                   jnp.bfloat16),
                               preferred_element_type=jnp.float32)
            out_ref[rows, lo:lo + width] = acc

        def ring_rdma(src, dst, sem_idx, dst_dev):
            return pltpu.make_async_remote_copy(
                src_ref=src, dst_ref=dst,
                send_sem=send_sems.at[sem_idx],
                recv_sem=recv_sems.at[sem_idx],
                device_id=(dst_dev,),
                device_id_type=pl.DeviceIdType.MESH,
            )

        colR = pl.ds(0, HC)
        colL = pl.ds(HC, HC)
        c_m1 = lax.rem(my - 1 + N_DEV, N_DEV)
        c_p1 = lax.rem(my + 1, N_DEV)
        c_p2 = lax.rem(my + 2, N_DEV)

        def rs_start(s, c_sR, c_sL):
            sbufR[s] = out_ref[pl.ds(c_sR * CHUNK, CHUNK), colR].astype(
                jnp.bfloat16)
            sbufL[s] = out_ref[pl.ds(c_sL * CHUNK, CHUNK), colL].astype(
                jnp.bfloat16)
            rR = ring_rdma(sbufR.at[s], rsR.at[s], s, right)
            rL = ring_rdma(sbufL.at[s], rsL.at[s], 3 + s, left)
            rR.start()
            rL.start()
            return rR, rL

        def rs_finish(rR, rL, s, c_rR, c_rL):
            rR.wait()
            rL.wait()
            out_ref[pl.ds(c_rR * CHUNK, CHUNK), colR] += rsR[s].astype(
                jnp.float32)
            out_ref[pl.ds(c_rL * CHUNK, CHUNK), colL] += rsL[s].astype(
                jnp.float32)

        partial_chunk(my, 0, H, with_shared=True)

        if _DO_RS:
            r0 = rs_start(0, my, my)
            partial_chunk(c_m1, 0, HC)
            partial_chunk(c_p1, HC, HC)
            rs_finish(*r0, 0, c_m1, c_p1)

            r1 = rs_start(1, c_m1, c_p1)
            partial_chunk(c_p2, 0, H)
            rs_finish(*r1, 1, c_p2, c_p2)

            r2 = rs_start(2, c_p2, c_p2)
            partial_chunk(c_p1, 0, HC)
            partial_chunk(c_m1, HC, HC)
            rs_finish(*r2, 2, c_p1, c_m1)
        else:
            partial_chunk(c_m1, 0, H)
            partial_chunk(c_p1, 0, H)
            partial_chunk(c_p2, 0, H)

        if not _DO_AG:
            return
        agR[3] = out_ref[pl.ds(c_p1 * CHUNK, CHUNK), colR].astype(
            jnp.bfloat16)
        agL[3] = out_ref[pl.ds(c_m1 * CHUNK, CHUNK), colL].astype(
            jnp.bfloat16)
        for s in range(N_DEV - 1):
            c_rR = lax.rem(my - s + N_DEV, N_DEV)
            c_rL = lax.rem(my + s, N_DEV)
            src_slot = 3 if s == 0 else s - 1
            rR = ring_rdma(agR.at[src_slot], agR.at[s], 6 + s, right)
            rL = ring_rdma(agL.at[src_slot], agL.at[s], 9 + s, left)
            rR.start()
            rL.start()
            rR.wait()
            rL.wait()
            out_ref[pl.ds(c_rR * CHUNK, CHUNK), colR] = agR[s].astype(
                jnp.float32)
            out_ref[pl.ds(c_rL * CHUNK, CHUNK), colL] = agL[s].astype(
                jnp.float32)

    out_shape = jax.ShapeDtypeStruct((N_TOK, H), jnp.float32)
    return pl.pallas_call(
        body,
        out_shape=out_shape,
        in_specs=[
            pl.BlockSpec(memory_space=pltpu.VMEM),
            pl.BlockSpec(memory_space=pltpu.VMEM),
            pl.BlockSpec(memory_space=pltpu.VMEM),
            pl.BlockSpec(memory_space=pltpu.VMEM),
            pl.BlockSpec(memory_space=pl.ANY),
            pl.BlockSpec(memory_space=pltpu.VMEM),
        ],
        out_specs=pl.BlockSpec(memory_space=pltpu.VMEM),
        scratch_shapes=[
            pltpu.VMEM((2, D, H), jnp.float32),
            pltpu.VMEM((SLOTS, H), jnp.bfloat16),
            pltpu.VMEM((3, CHUNK, HC), jnp.bfloat16),
            pltpu.VMEM((3, CHUNK, HC), jnp.bfloat16),
            pltpu.VMEM((3, CHUNK, HC), jnp.bfloat16),
            pltpu.VMEM((3, CHUNK, HC), jnp.bfloat16),
            pltpu.VMEM((4, CHUNK, HC), jnp.bfloat16),
            pltpu.VMEM((4, CHUNK, HC), jnp.bfloat16),
            pltpu.SemaphoreType.DMA((2,)),
            pltpu.SemaphoreType.DMA((12,)),
            pltpu.SemaphoreType.DMA((12,)),
        ],
        compiler_params=pltpu.CompilerParams(
            collective_id=0,
            vmem_limit_bytes=100 * 1024 * 1024,
        ),
    )(x, col2d, colrow, pw2d, expert_W, shared_W)


# device time: 88837 ns/iter; 1.1183x vs baseline; 1.0763x over previous
import os

import jax
import jax.numpy as jnp
from jax import lax
from jax.experimental import pallas as pl
from jax.experimental.pallas import tpu as pltpu

_ABLATE = os.environ.get("KERNEL_ABLATE", "full")
_DO_COMPUTE = _ABLATE != "commonly"
_DO_RS = _ABLATE not in ("nocomm",)
_DO_AG = _ABLATE not in ("nocomm", "noag")

N_DEV = 4
N_TOK = 2048
D = 1024
H = 1024
E_LOCAL = 8
CAP = 128
SLOTS = E_LOCAL * CAP
CHUNK = N_TOK // N_DEV
HC = H // 2
QC = HC // 2


def kernel(x, router_W, route_idx, expert_W, shared_W):
    first = lax.axis_index("i") * E_LOCAL

    scores = x @ router_W
    s_max = jnp.max(scores, axis=-1, keepdims=True)
    p_top = 1.0 / jnp.sum(jnp.exp(scores - s_max), axis=-1)
    ridx = route_idx[:, 0]
    lej = ridx - first
    mine = (lej >= 0) & (lej < E_LOCAL)
    emask = (lej[:, None] == jnp.arange(E_LOCAL)[None, :])
    lt = jnp.tri(N_TOK, dtype=jnp.bfloat16)
    pos = (lt @ emask.astype(jnp.bfloat16)).astype(jnp.int32) - 1
    pos_tok = jnp.sum(jnp.where(emask, pos, 0), axis=-1)
    col = jnp.where(mine & (pos_tok < CAP), lej * CAP + pos_tok, SLOTS)
    col2d = col[:, None].astype(jnp.int32)
    pw2d = p_top[:, None].astype(jnp.float32)

    def body(x_ref, col_in_ref, pw_in_ref, expert_W_ref, shared_W_ref,
             out_ref, wbuf, ys_ref, sbufR, sbufL, rsR, rsL,
             agR, agL, dma_sems, send_sems, recv_sems):
        my = lax.axis_index("i")
        left = lax.rem(my - 1 + N_DEV, N_DEV)
        right = lax.rem(my + 1, N_DEV)
        col_ref = col_in_ref
        pw_ref = pw_in_ref

        barrier_sem = pltpu.get_barrier_semaphore()
        for nbr in (left, right):
            pl.semaphore_signal(
                barrier_sem, inc=1,
                device_id=(nbr,), device_id_type=pl.DeviceIdType.MESH,
            )
        pl.semaphore_wait(barrier_sem, 2)

        for j in range(4):
            pltpu.make_async_copy(expert_W_ref.at[j], wbuf.at[j],
                                  dma_sems.at[j]).start()

        xb = x_ref[:, :].astype(jnp.bfloat16)

        slot_i = lax.broadcasted_iota(jnp.int32, (N_TOK, SLOTS), 1)
        onehot = (slot_i == col_ref[:, :]).astype(jnp.bfloat16)

        xg = lax.dot_general(onehot, xb, (((0,), (0,)), ((), ())),
                             preferred_element_type=jnp.float32)
        xgb = xg.astype(jnp.bfloat16)

        for j in range(E_LOCAL):
            pltpu.make_async_copy(expert_W_ref.at[j], wbuf.at[j % 4],
                                  dma_sems.at[j % 4]).wait()
            if _DO_COMPUTE:
                yj = jnp.dot(xgb[j * CAP:(j + 1) * CAP, :],
                             wbuf[j % 4].astype(jnp.bfloat16),
                             preferred_element_type=jnp.float32)
                ys_ref[j * CAP:(j + 1) * CAP, :] = yj.astype(jnp.bfloat16)
            else:
                ys_ref[j * CAP:(j + 1) * CAP, :] = (
                    wbuf[j % 4, :CAP, :].astype(jnp.bfloat16))
            if j + 4 < E_LOCAL:
                pltpu.make_async_copy(expert_W_ref.at[j + 4], wbuf.at[j % 4],
                                      dma_sems.at[j % 4]).start()
        ys = ys_ref[:, :]

        def partial_chunk(c, lo, width, with_shared=False):
            rows = pl.ds(c * CHUNK, CHUNK)
            slot_j = lax.broadcasted_iota(jnp.int32, (CHUNK, SLOTS), 1)
            Sw = jnp.where(slot_j == col_ref[rows, :], pw_ref[rows, :],
                           0.0).astype(jnp.bfloat16)
            acc = jnp.dot(Sw, ys[:, lo:lo + width],
                          preferred_element_type=jnp.float32)
            if with_shared:
                acc += jnp.dot(x_ref[rows, :].astype(jnp.bfloat16),
                               shared_W_ref[:, lo:lo + width].astype(
                                   jnp.bfloat16),
                               preferred_element_type=jnp.float32)
            out_ref[rows, lo:lo + width] = acc

        def ring_rdma(src, dst, sem_idx, dst_dev):
            return pltpu.make_async_remote_copy(
                src_ref=src, dst_ref=dst,
                send_sem=send_sems.at[sem_idx],
                recv_sem=recv_sems.at[sem_idx],
                device_id=(dst_dev,),
                device_id_type=pl.DeviceIdType.MESH,
            )

        colR = pl.ds(0, HC)
        colL = pl.ds(HC, HC)
        c_m1 = lax.rem(my - 1 + N_DEV, N_DEV)
        c_p1 = lax.rem(my + 1, N_DEV)
        c_p2 = lax.rem(my + 2, N_DEV)
        subs = (pl.ds(0, QC), pl.ds(QC, QC))

        def rs_start(s, c_sR, c_sL):
            sbufR[s] = out_ref[pl.ds(c_sR * CHUNK, CHUNK), colR].astype(
                jnp.bfloat16)
            sbufL[s] = out_ref[pl.ds(c_sL * CHUNK, CHUNK), colL].astype(
                jnp.bfloat16)
            rdmas = []
            for u, sub in enumerate(subs):
                rR = ring_rdma(sbufR.at[s, :, sub], rsR.at[s, :, sub],
                               2 * s + u, right)
                rL = ring_rdma(sbufL.at[s, :, sub], rsL.at[s, :, sub],
                               6 + 2 * s + u, left)
                rR.start()
                rL.start()
                rdmas.append((rR, rL))
            return rdmas

        def rs_finish(rdmas, s, c_rR, c_rL):
            rowsR = pl.ds(c_rR * CHUNK, CHUNK)
            rowsL = pl.ds(c_rL * CHUNK, CHUNK)
            for u, sub in enumerate(subs):
                rR, rL = rdmas[u]
                rR.wait_recv()
                rL.wait_recv()
                out_ref[rowsR, pl.ds(u * QC, QC)] += rsR[
                    s, :, sub].astype(jnp.float32)
                out_ref[rowsL, pl.ds(HC + u * QC, QC)] += rsL[
                    s, :, sub].astype(jnp.float32)
            for rR, rL in rdmas:
                rR.wait_send()
                rL.wait_send()

        partial_chunk(my, 0, H, with_shared=True)

        if _DO_RS:
            r0 = rs_start(0, my, my)
            partial_chunk(c_m1, 0, HC)
            partial_chunk(c_p1, HC, HC)
            rs_finish(r0, 0, c_m1, c_p1)

            r1 = rs_start(1, c_m1, c_p1)
            partial_chunk(c_p2, 0, H)
            rs_finish(r1, 1, c_p2, c_p2)

            r2 = rs_start(2, c_p2, c_p2)
            partial_chunk(c_p1, 0, HC)
            partial_chunk(c_m1, HC, HC)
            rs_finish(r2, 2, c_p1, c_m1)
        else:
            partial_chunk(c_m1, 0, H)
            partial_chunk(c_p1, 0, H)
            partial_chunk(c_p2, 0, H)

        if not _DO_AG:
            return
        agR[3] = out_ref[pl.ds(c_p1 * CHUNK, CHUNK), colR].astype(
            jnp.bfloat16)
        agL[3] = out_ref[pl.ds(c_m1 * CHUNK, CHUNK), colL].astype(
            jnp.bfloat16)
        ag_rdmas = {}
        pending_store = None
        for s in range(N_DEV - 1):
            src_slot = 3 if s == 0 else s - 1
            for u, sub in enumerate(subs):
                if s > 0:
                    rR_p, rL_p = ag_rdmas[(s - 1, u)]
                    rR_p.wait_recv()
                    rL_p.wait_recv()
                rR = ring_rdma(agR.at[src_slot, :, sub], agR.at[s, :, sub],
                               12 + 2 * s + u, right)
                rL = ring_rdma(agL.at[src_slot, :, sub], agL.at[s, :, sub],
                               18 + 2 * s + u, left)
                rR.start()
                rL.start()
                ag_rdmas[(s, u)] = (rR, rL)
                if pending_store is not None:
                    pending_store()
                    pending_store = None
                if s > 0:
                    ps, pu = s - 1, u

                    def _store(ps=ps, pu=pu):
                        c_rR = lax.rem(my - ps + N_DEV, N_DEV)
                        c_rL = lax.rem(my + ps, N_DEV)
                        psub = subs[pu]
                        out_ref[pl.ds(c_rR * CHUNK, CHUNK),
                                pl.ds(pu * QC, QC)] = agR[
                            ps, :, psub].astype(jnp.float32)
                        out_ref[pl.ds(c_rL * CHUNK, CHUNK),
                                pl.ds(HC + pu * QC, QC)] = agL[
                            ps, :, psub].astype(jnp.float32)
                    pending_store = _store
        if pending_store is not None:
            pending_store()
        s_last = N_DEV - 2
        for u, sub in enumerate(subs):
            rR, rL = ag_rdmas[(s_last, u)]
            rR.wait_recv()
            rL.wait_recv()
            c_rR = lax.rem(my - s_last + N_DEV, N_DEV)
            c_rL = lax.rem(my + s_last, N_DEV)
            out_ref[pl.ds(c_rR * CHUNK, CHUNK), pl.ds(u * QC, QC)] = agR[
                s_last, :, sub].astype(jnp.float32)
            out_ref[pl.ds(c_rL * CHUNK, CHUNK),
                    pl.ds(HC + u * QC, QC)] = agL[
                s_last, :, sub].astype(jnp.float32)
        for (rR, rL) in ag_rdmas.values():
            rR.wait_send()
            rL.wait_send()

    out_shape = jax.ShapeDtypeStruct((N_TOK, H), jnp.float32)
    return pl.pallas_call(
        body,
        out_shape=out_shape,
        in_specs=[
            pl.BlockSpec(memory_space=pltpu.VMEM),
            pl.BlockSpec(memory_space=pltpu.VMEM),
            pl.BlockSpec(memory_space=pltpu.VMEM),
            pl.BlockSpec(memory_space=pl.ANY),
            pl.BlockSpec(memory_space=pltpu.VMEM),
        ],
        out_specs=pl.BlockSpec(memory_space=pltpu.VMEM),
        scratch_shapes=[
            pltpu.VMEM((4, D, H), jnp.float32),
            pltpu.VMEM((SLOTS, H), jnp.bfloat16),
            pltpu.VMEM((3, CHUNK, HC), jnp.bfloat16),
            pltpu.VMEM((3, CHUNK, HC), jnp.bfloat16),
            pltpu.VMEM((3, CHUNK, HC), jnp.bfloat16),
            pltpu.VMEM((3, CHUNK, HC), jnp.bfloat16),
            pltpu.VMEM((4, CHUNK, HC), jnp.bfloat16),
            pltpu.VMEM((4, CHUNK, HC), jnp.bfloat16),
            pltpu.SemaphoreType.DMA((4,)),
            pltpu.SemaphoreType.DMA((24,)),
            pltpu.SemaphoreType.DMA((24,)),
        ],
        compiler_params=pltpu.CompilerParams(
            collective_id=0,
            vmem_limit_bytes=100 * 1024 * 1024,
        ),
    )(x, col2d, pw2d, expert_W, shared_W)
